# baseline (speedup 1.0000x reference)
import functools

import jax
import jax.numpy as jnp
from jax import lax
from jax.experimental import pallas as pl
from jax.experimental.pallas import tpu as pltpu
from jax.experimental.pallas import tpu_sc as plsc

EMBED = 64
SCALE = 8.0

_info = plsc.get_sparse_core_info()
_NC, _NS, _L = _info.num_cores, _info.num_subcores, _info.num_lanes
_NW = _NC * _NS


@jax.jit
def _lookup(x, table):
    n_rows, row_len = x.shape
    rows_per_w = n_rows // _NW
    mesh = plsc.VectorSubcoreMesh(core_axis_name="c", subcore_axis_name="s")

    @functools.partial(
        pl.kernel,
        out_type=jax.ShapeDtypeStruct((n_rows, row_len, EMBED), jnp.float32),
        mesh=mesh,
        scratch_types=[
            pltpu.VMEM((row_len,), jnp.int32),
            pltpu.VMEM((row_len, EMBED), jnp.float32),
            pltpu.SemaphoreType.DMA,
        ],
        compiler_params=pltpu.CompilerParams(use_tc_tiling_on_sc=False),
    )
    def k(x_hbm, table_hbm, out_hbm, idx_v, rows_v, sem):
        wid = lax.axis_index("s") * _NC + lax.axis_index("c")
        base = wid * rows_per_w

        def row_body(g, carry):
            r = base + g
            pltpu.sync_copy(x_hbm.at[r], idx_v)
            pltpu.async_copy(table_hbm.at[idx_v], rows_v, sem).wait()

            def scale_j(j, c2):
                for c in range(EMBED // _L):
                    sl = pl.ds(c * _L, _L)
                    rows_v[j, sl] = rows_v[j, sl] * SCALE
                return c2

            lax.fori_loop(0, row_len, scale_j, 0)
            pltpu.sync_copy(rows_v, out_hbm.at[r])
            return carry

        lax.fori_loop(0, rows_per_w, row_body, 0)

    return k(x, table)


def kernel(x, embedding_table):
    return _lookup(x.astype(jnp.int32), embedding_table)
